# baseline (speedup 1.0000x reference)
import functools

import jax
import jax.numpy as jnp
from jax import lax
from jax.experimental import pallas as pl
from jax.experimental.pallas import tpu as pltpu
from jax.experimental.pallas import tpu_sc as plsc

NC = 2
NS = 16
NW = NC * NS

CHUNK = 1
NBUF = 8
LANES = 16
UNROLL = 16
NACC = 4


def _row_stats(bufs, b, r, vocab):
    n_iter = vocab // (LANES * UNROLL)

    def maxstep(k, accs):
        accs = list(accs)
        for u in range(UNROLL):
            v = bufs[b, r, pl.ds(k * (LANES * UNROLL) + u * LANES, LANES)]
            a = u % NACC
            accs[a] = jnp.maximum(accs[a], v)
        return tuple(accs)

    neg = jnp.full((LANES,), -1e30, dtype=jnp.float32)
    maccs = lax.fori_loop(0, n_iter, maxstep, (neg,) * NACC)
    mf = maccs[0]
    for a in range(1, NACC):
        mf = jnp.maximum(mf, maccs[a])

    def sumstep(k, accs):
        accs = list(accs)
        for u in range(UNROLL):
            v = bufs[b, r, pl.ds(k * (LANES * UNROLL) + u * LANES, LANES)]
            a = u % NACC
            accs[a] = accs[a] + jnp.exp(v - mf)
        return tuple(accs)

    zero = jnp.zeros((LANES,), dtype=jnp.float32)
    saccs = lax.fori_loop(0, n_iter, sumstep, (zero,) * NACC)
    sf = saccs[0]
    for a in range(1, NACC):
        sf = sf + saccs[a]
    return mf, sf


def _sc_body(table, idx2d, tgt, out, om, os_, ot,
             idx_v, tgt_v, bufs, sm_v, ss_v, st_v, gsems, ssems,
             *, n_chunks, vocab):
    wid = lax.axis_index("s") * NC + lax.axis_index("c")
    base = wid * n_chunks
    rows_per_w = n_chunks * CHUNK
    pltpu.sync_copy(idx2d.at[pl.ds(base, n_chunks)], idx_v)
    pltpu.sync_copy(tgt.at[pl.ds(base * LANES, n_chunks * LANES)], tgt_v)
    lane = lax.iota(jnp.int32, LANES)

    def gather_start(b, c):
        return pltpu.async_copy(table.at[idx_v.at[c]], bufs.at[b],
                                gsems.at[b])

    def gather_wait(b):
        pltpu.make_async_copy(out.at[pl.ds(0, CHUNK)], bufs.at[b],
                              gsems.at[b]).wait()

    def scatter_start(b, c):
        return pltpu.async_copy(bufs.at[b],
                                out.at[pl.ds((base + c) * CHUNK, CHUNK)],
                                ssems.at[b])

    def scatter_wait(b):
        pltpu.make_async_copy(bufs.at[b], out.at[pl.ds(0, CHUNK)],
                              ssems.at[b]).wait()

    def compute(b, c, t_acc):
        tv = tgt_v[pl.ds(c * LANES, LANES)]
        for r in range(CHUNK):
            row_local = c * CHUNK + r
            mf, sf = _row_stats(bufs, b, r, vocab)
            sm_v[pl.ds(row_local * LANES, LANES)] = mf
            ss_v[pl.ds(row_local * LANES, LANES)] = sf
            t = tv[r]
            ta = (t // LANES) * LANES
            v = bufs[b, r, pl.ds(ta, LANES)]
            t_acc = t_acc + jnp.where(lane + ta == t, v, 0.0)
        return t_acc

    for b in range(NBUF):
        gather_start(b, b)

    t_acc = jnp.zeros((LANES,), jnp.float32)

    def group(g, t_acc):
        for b in range(NBUF):
            c = g * NBUF + b
            gather_wait(b)
            scatter_start(b, c)
            t_acc = compute(b, c, t_acc)
            scatter_wait(b)
            gather_start(b, c + NBUF)
        return t_acc

    n_groups = n_chunks // NBUF
    t_acc = lax.fori_loop(0, n_groups - 1, group, t_acc)

    for b in range(NBUF):
        c = n_chunks - NBUF + b
        gather_wait(b)
        scatter_start(b, c)
        t_acc = compute(b, c, t_acc)
        scatter_wait(b)

    st_v[...] = t_acc
    pltpu.sync_copy(sm_v, om.at[wid])
    pltpu.sync_copy(ss_v, os_.at[wid])
    pltpu.sync_copy(st_v, ot.at[wid])


def _sc_gather_ce(table, idx2d, flat_tgt, vocab):
    n = idx2d.shape[0] * CHUNK
    n_chunks = n // (NW * CHUNK)
    rows_per_w = n_chunks * CHUNK
    mesh = plsc.VectorSubcoreMesh(core_axis_name="c", subcore_axis_name="s")
    kern = functools.partial(
        pl.kernel,
        mesh=mesh,
        out_type=[
            jax.ShapeDtypeStruct((n, vocab), jnp.float32),
            jax.ShapeDtypeStruct((NW, rows_per_w * LANES), jnp.float32),
            jax.ShapeDtypeStruct((NW, rows_per_w * LANES), jnp.float32),
            jax.ShapeDtypeStruct((NW, LANES), jnp.float32),
        ],
        scratch_types=[
            pltpu.VMEM((n_chunks, CHUNK), jnp.int32),
            pltpu.VMEM((n_chunks * LANES,), jnp.int32),
            pltpu.VMEM((NBUF, CHUNK, vocab), jnp.float32),
            pltpu.VMEM((rows_per_w * LANES,), jnp.float32),
            pltpu.VMEM((rows_per_w * LANES,), jnp.float32),
            pltpu.VMEM((LANES,), jnp.float32),
            pltpu.SemaphoreType.DMA((NBUF,)),
            pltpu.SemaphoreType.DMA((NBUF,)),
        ],
    )(functools.partial(_sc_body, n_chunks=n_chunks, vocab=vocab))
    return kern(table, idx2d, flat_tgt)


def _finish_body(m_ref, s_ref, t_ref, loss_ref, *, n):
    m = m_ref[...]
    s = s_ref[...]
    mm = jnp.max(m, axis=1, keepdims=True)
    se = jnp.sum(s * jnp.exp(m - mm), axis=1)
    lse_sum = jnp.sum(jnp.log(se) + mm[:, 0])
    loss_ref[0, 0] = (lse_sum - jnp.sum(t_ref[...])) / n


def _prep_targets(flat_tgt, n):
    t2 = jnp.zeros((n // CHUNK, LANES), jnp.int32)
    t2 = t2.at[:, :CHUNK].set(flat_tgt.reshape(n // CHUNK, CHUNK))
    return t2.reshape(-1)


def _finish_loss(om, os_, ot, n):
    loss = pl.pallas_call(
        functools.partial(_finish_body, n=n),
        grid=(1,),
        in_specs=[pl.BlockSpec(om.shape, lambda i: (0, 0)),
                  pl.BlockSpec(os_.shape, lambda i: (0, 0)),
                  pl.BlockSpec(ot.shape, lambda i: (0, 0))],
        out_specs=pl.BlockSpec((1, 1), lambda i: (0, 0),
                               memory_space=pltpu.SMEM),
        out_shape=jax.ShapeDtypeStruct((1, 1), jnp.float32),
    )(om, os_, ot)
    return loss[0, 0]


def kernel(indices, targets, table):
    B, T = indices.shape
    vocab = table.shape[1]
    n = B * T
    flat_idx = indices.reshape(n).astype(jnp.int32)
    flat_tgt = targets.reshape(n).astype(jnp.int32)
    idx2d = flat_idx.reshape(n // CHUNK, CHUNK)
    tgt16 = _prep_targets(flat_tgt, n)

    logits_flat, om, os_, ot = _sc_gather_ce(table, idx2d, tgt16, vocab)
    loss = _finish_loss(om.reshape(n, LANES), os_.reshape(n, LANES), ot, n)
    return logits_flat.reshape(B, T, vocab), loss


# speedup vs baseline: 1.0238x; 1.0238x over previous
import functools

import jax
import jax.numpy as jnp
from jax import lax
from jax.experimental import pallas as pl
from jax.experimental.pallas import tpu as pltpu
from jax.experimental.pallas import tpu_sc as plsc

NC = 2
NS = 16
NW = NC * NS

CHUNK = 2
NBUF = 4
LANES = 16
UNROLL = 16
NACC = 4


def _row_stats(bufs, b, r, vocab):
    n_iter = vocab // (LANES * UNROLL)

    def maxstep(k, accs):
        accs = list(accs)
        for u in range(UNROLL):
            v = bufs[b, r, pl.ds(k * (LANES * UNROLL) + u * LANES, LANES)]
            a = u % NACC
            accs[a] = jnp.maximum(accs[a], v)
        return tuple(accs)

    neg = jnp.full((LANES,), -1e30, dtype=jnp.float32)
    maccs = lax.fori_loop(0, n_iter, maxstep, (neg,) * NACC)
    mf = maccs[0]
    for a in range(1, NACC):
        mf = jnp.maximum(mf, maccs[a])

    def sumstep(k, accs):
        accs = list(accs)
        for u in range(UNROLL):
            v = bufs[b, r, pl.ds(k * (LANES * UNROLL) + u * LANES, LANES)]
            a = u % NACC
            accs[a] = accs[a] + jnp.exp(v - mf)
        return tuple(accs)

    zero = jnp.zeros((LANES,), dtype=jnp.float32)
    saccs = lax.fori_loop(0, n_iter, sumstep, (zero,) * NACC)
    sf = saccs[0]
    for a in range(1, NACC):
        sf = sf + saccs[a]
    return mf, sf


def _sc_body(table, idx2d, tgt, out, om, os_, ot,
             idx_v, tgt_v, bufs, sm_v, ss_v, st_v, gsems, ssems,
             *, n_chunks, vocab):
    wid = lax.axis_index("s") * NC + lax.axis_index("c")
    base = wid * n_chunks
    pltpu.sync_copy(idx2d.at[pl.ds(base, n_chunks)], idx_v)
    pltpu.sync_copy(tgt.at[pl.ds(base * LANES, n_chunks * LANES)], tgt_v)
    lane = lax.iota(jnp.int32, LANES)

    def gather_start(b, c):
        return pltpu.async_copy(table.at[idx_v.at[c]], bufs.at[b],
                                gsems.at[b])

    def gather_wait(b):
        pltpu.make_async_copy(out.at[pl.ds(0, CHUNK)], bufs.at[b],
                              gsems.at[b]).wait()

    def scatter_start(b, c):
        return pltpu.async_copy(bufs.at[b],
                                out.at[pl.ds((base + c) * CHUNK, CHUNK)],
                                ssems.at[b])

    def scatter_wait(b):
        pltpu.make_async_copy(bufs.at[b], out.at[pl.ds(0, CHUNK)],
                              ssems.at[b]).wait()

    def compute(b, c, t_acc):
        tv = tgt_v[pl.ds(c * LANES, LANES)]
        for r in range(CHUNK):
            row_local = c * CHUNK + r
            mf, sf = _row_stats(bufs, b, r, vocab)
            sm_v[pl.ds(row_local * LANES, LANES)] = mf
            ss_v[pl.ds(row_local * LANES, LANES)] = sf
            t = tv[r]
            ta = (t // LANES) * LANES
            v = bufs[b, r, pl.ds(ta, LANES)]
            t_acc = t_acc + jnp.where(lane + ta == t, v, 0.0)
        return t_acc

    for b in range(NBUF):
        gather_start(b, b)

    t_acc = jnp.zeros((LANES,), jnp.float32)

    def group(g, t_acc):
        for b in range(NBUF):
            c = g * NBUF + b
            gather_wait(b)
            scatter_start(b, c)
            t_acc = compute(b, c, t_acc)
            scatter_wait(b)
            gather_start(b, c + NBUF)
        return t_acc

    n_groups = n_chunks // NBUF
    t_acc = lax.fori_loop(0, n_groups - 1, group, t_acc)

    for b in range(NBUF):
        c = n_chunks - NBUF + b
        gather_wait(b)
        scatter_start(b, c)
        t_acc = compute(b, c, t_acc)
        scatter_wait(b)

    st_v[...] = t_acc
    pltpu.sync_copy(sm_v, om.at[wid])
    pltpu.sync_copy(ss_v, os_.at[wid])
    pltpu.sync_copy(st_v, ot.at[wid])


def _sc_gather_ce(table, idx2d, flat_tgt, vocab):
    n = idx2d.shape[0] * CHUNK
    n_chunks = n // (NW * CHUNK)
    rows_per_w = n_chunks * CHUNK
    mesh = plsc.VectorSubcoreMesh(core_axis_name="c", subcore_axis_name="s")
    kern = functools.partial(
        pl.kernel,
        mesh=mesh,
        out_type=[
            jax.ShapeDtypeStruct((n, vocab), jnp.float32),
            jax.ShapeDtypeStruct((NW, rows_per_w * LANES), jnp.float32),
            jax.ShapeDtypeStruct((NW, rows_per_w * LANES), jnp.float32),
            jax.ShapeDtypeStruct((NW, LANES), jnp.float32),
        ],
        scratch_types=[
            pltpu.VMEM((n_chunks, CHUNK), jnp.int32),
            pltpu.VMEM((n_chunks * LANES,), jnp.int32),
            pltpu.VMEM((NBUF, CHUNK, vocab), jnp.float32),
            pltpu.VMEM((rows_per_w * LANES,), jnp.float32),
            pltpu.VMEM((rows_per_w * LANES,), jnp.float32),
            pltpu.VMEM((LANES,), jnp.float32),
            pltpu.SemaphoreType.DMA((NBUF,)),
            pltpu.SemaphoreType.DMA((NBUF,)),
        ],
    )(functools.partial(_sc_body, n_chunks=n_chunks, vocab=vocab))
    return kern(table, idx2d, flat_tgt)


def _finish_body(m_ref, s_ref, t_ref, loss_ref, *, n):
    m = m_ref[...]
    s = s_ref[...]
    mm = jnp.max(m, axis=1, keepdims=True)
    se = jnp.sum(s * jnp.exp(m - mm), axis=1)
    lse_sum = jnp.sum(jnp.log(se) + mm[:, 0])
    loss_ref[0, 0] = (lse_sum - jnp.sum(t_ref[...])) / n


def _prep_targets(flat_tgt, n):
    t2 = jnp.zeros((n // CHUNK, LANES), jnp.int32)
    t2 = t2.at[:, :CHUNK].set(flat_tgt.reshape(n // CHUNK, CHUNK))
    return t2.reshape(-1)


def _finish_loss(om, os_, ot, n):
    loss = pl.pallas_call(
        functools.partial(_finish_body, n=n),
        grid=(1,),
        in_specs=[pl.BlockSpec(om.shape, lambda i: (0, 0)),
                  pl.BlockSpec(os_.shape, lambda i: (0, 0)),
                  pl.BlockSpec(ot.shape, lambda i: (0, 0))],
        out_specs=pl.BlockSpec((1, 1), lambda i: (0, 0),
                               memory_space=pltpu.SMEM),
        out_shape=jax.ShapeDtypeStruct((1, 1), jnp.float32),
    )(om, os_, ot)
    return loss[0, 0]


def kernel(indices, targets, table):
    B, T = indices.shape
    vocab = table.shape[1]
    n = B * T
    flat_idx = indices.reshape(n).astype(jnp.int32)
    flat_tgt = targets.reshape(n).astype(jnp.int32)
    idx2d = flat_idx.reshape(n // CHUNK, CHUNK)
    tgt16 = _prep_targets(flat_tgt, n)

    logits_flat, om, os_, ot = _sc_gather_ce(table, idx2d, tgt16, vocab)
    loss = _finish_loss(om.reshape(n, LANES), os_.reshape(n, LANES), ot, n)
    return logits_flat.reshape(B, T, vocab), loss


# speedup vs baseline: 1.0265x; 1.0026x over previous
import functools

import jax
import jax.numpy as jnp
from jax import lax
from jax.experimental import pallas as pl
from jax.experimental.pallas import tpu as pltpu
from jax.experimental.pallas import tpu_sc as plsc

NC = 2
NS = 16
NW = NC * NS

CHUNK = 2
NBUF = 4
LANES = 16
UNROLL = 16
NACC = 4


def _row_stats(bufs, b, r, vocab):
    n_iter = vocab // (LANES * UNROLL)

    def maxstep(k, accs):
        accs = list(accs)
        for u in range(UNROLL):
            v = bufs[b, r, pl.ds(k * (LANES * UNROLL) + u * LANES, LANES)]
            a = u % NACC
            accs[a] = jnp.maximum(accs[a], v)
        return tuple(accs)

    neg = jnp.full((LANES,), -1e30, dtype=jnp.float32)
    maccs = lax.fori_loop(0, n_iter, maxstep, (neg,) * NACC)
    mf = maccs[0]
    for a in range(1, NACC):
        mf = jnp.maximum(mf, maccs[a])

    def sumstep(k, accs):
        accs = list(accs)
        for u in range(UNROLL):
            v = bufs[b, r, pl.ds(k * (LANES * UNROLL) + u * LANES, LANES)]
            a = u % NACC
            accs[a] = accs[a] + jnp.exp(v - mf)
        return tuple(accs)

    zero = jnp.zeros((LANES,), dtype=jnp.float32)
    saccs = lax.fori_loop(0, n_iter, sumstep, (zero,) * NACC)
    sf = saccs[0]
    for a in range(1, NACC):
        sf = sf + saccs[a]
    return mf, sf


def _sc_body(table, idx2d, tgt, out, om, os_, ot,
             idx_v, tgt_v, bufs, sm_v, ss_v, st_v, gsems, ssems,
             *, n_chunks, vocab):
    wid = lax.axis_index("s") * NC + lax.axis_index("c")
    base = wid * n_chunks
    pltpu.sync_copy(idx2d.at[pl.ds(base, n_chunks)], idx_v)
    pltpu.sync_copy(tgt.at[pl.ds(base * LANES, n_chunks * LANES)], tgt_v)
    lane = lax.iota(jnp.int32, LANES)

    def gather_start(b, c):
        return pltpu.async_copy(table.at[idx_v.at[c]], bufs.at[b],
                                gsems.at[b])

    def gather_wait(b, c):
        pltpu.make_async_copy(table.at[idx_v.at[c]], bufs.at[b],
                              gsems.at[b]).wait()

    def scatter_start(b, c):
        return pltpu.async_copy(bufs.at[b],
                                out.at[pl.ds((base + c) * CHUNK, CHUNK)],
                                ssems.at[b])

    def scatter_wait(b, c):
        pltpu.make_async_copy(bufs.at[b],
                              out.at[pl.ds((base + c) * CHUNK, CHUNK)],
                              ssems.at[b]).wait()

    def compute(b, c, t_acc):
        tv = tgt_v[pl.ds(c * LANES, LANES)]
        for r in range(CHUNK):
            row_local = c * CHUNK + r
            mf, sf = _row_stats(bufs, b, r, vocab)
            sm_v[pl.ds(row_local * LANES, LANES)] = mf
            ss_v[pl.ds(row_local * LANES, LANES)] = sf
            t = tv[r]
            ta = (t // LANES) * LANES
            v = bufs[b, r, pl.ds(ta, LANES)]
            t_acc = t_acc + jnp.where(lane + ta == t, v, 0.0)
        return t_acc

    for b in range(NBUF):
        gather_start(b, b)

    t_acc = jnp.zeros((LANES,), jnp.float32)

    def group(g, t_acc):
        for b in range(NBUF):
            c = g * NBUF + b
            gather_wait(b, c)
            scatter_start(b, c)
            t_acc = compute(b, c, t_acc)
            scatter_wait(b, c)
            gather_start(b, c + NBUF)
        return t_acc

    n_groups = n_chunks // NBUF
    t_acc = lax.fori_loop(0, n_groups - 1, group, t_acc)

    for b in range(NBUF):
        c = n_chunks - NBUF + b
        gather_wait(b, c)
        scatter_start(b, c)
        t_acc = compute(b, c, t_acc)
        scatter_wait(b, c)

    st_v[...] = t_acc
    pltpu.sync_copy(sm_v, om.at[wid])
    pltpu.sync_copy(ss_v, os_.at[wid])
    pltpu.sync_copy(st_v, ot.at[wid])


def _sc_gather_ce(table, idx2d, flat_tgt, vocab):
    n = idx2d.shape[0] * CHUNK
    n_chunks = n // (NW * CHUNK)
    rows_per_w = n_chunks * CHUNK
    mesh = plsc.VectorSubcoreMesh(core_axis_name="c", subcore_axis_name="s")
    kern = functools.partial(
        pl.kernel,
        mesh=mesh,
        out_type=[
            jax.ShapeDtypeStruct((n, vocab), jnp.float32),
            jax.ShapeDtypeStruct((NW, rows_per_w * LANES), jnp.float32),
            jax.ShapeDtypeStruct((NW, rows_per_w * LANES), jnp.float32),
            jax.ShapeDtypeStruct((NW, LANES), jnp.float32),
        ],
        scratch_types=[
            pltpu.VMEM((n_chunks, CHUNK), jnp.int32),
            pltpu.VMEM((n_chunks * LANES,), jnp.int32),
            pltpu.VMEM((NBUF, CHUNK, vocab), jnp.float32),
            pltpu.VMEM((rows_per_w * LANES,), jnp.float32),
            pltpu.VMEM((rows_per_w * LANES,), jnp.float32),
            pltpu.VMEM((LANES,), jnp.float32),
            pltpu.SemaphoreType.DMA((NBUF,)),
            pltpu.SemaphoreType.DMA((NBUF,)),
        ],
    )(functools.partial(_sc_body, n_chunks=n_chunks, vocab=vocab))
    return kern(table, idx2d, flat_tgt)


def _finish_body(m_ref, s_ref, t_ref, loss_ref, *, n):
    m = m_ref[...]
    s = s_ref[...]
    mm = jnp.max(m, axis=1, keepdims=True)
    se = jnp.sum(s * jnp.exp(m - mm), axis=1)
    lse_sum = jnp.sum(jnp.log(se) + mm[:, 0])
    loss_ref[0, 0] = (lse_sum - jnp.sum(t_ref[...])) / n


def _prep_targets(flat_tgt, n):
    t2 = jnp.zeros((n // CHUNK, LANES), jnp.int32)
    t2 = t2.at[:, :CHUNK].set(flat_tgt.reshape(n // CHUNK, CHUNK))
    return t2.reshape(-1)


def _finish_loss(om, os_, ot, n):
    loss = pl.pallas_call(
        functools.partial(_finish_body, n=n),
        grid=(1,),
        in_specs=[pl.BlockSpec(om.shape, lambda i: (0, 0)),
                  pl.BlockSpec(os_.shape, lambda i: (0, 0)),
                  pl.BlockSpec(ot.shape, lambda i: (0, 0))],
        out_specs=pl.BlockSpec((1, 1), lambda i: (0, 0),
                               memory_space=pltpu.SMEM),
        out_shape=jax.ShapeDtypeStruct((1, 1), jnp.float32),
    )(om, os_, ot)
    return loss[0, 0]


def kernel(indices, targets, table):
    B, T = indices.shape
    vocab = table.shape[1]
    n = B * T
    flat_idx = indices.reshape(n).astype(jnp.int32)
    flat_tgt = targets.reshape(n).astype(jnp.int32)
    idx2d = flat_idx.reshape(n // CHUNK, CHUNK)
    tgt16 = _prep_targets(flat_tgt, n)

    logits_flat, om, os_, ot = _sc_gather_ce(table, idx2d, tgt16, vocab)
    loss = _finish_loss(om.reshape(n, LANES), os_.reshape(n, LANES), ot, n)
    return logits_flat.reshape(B, T, vocab), loss
